# baseline (speedup 1.0000x reference)
import jax
import jax.numpy as jnp
from jax import lax
from jax.experimental import pallas as pl
from jax.experimental.pallas import tpu as pltpu
from jax.experimental.pallas import tpu_sc as plsc

BATCH = 4096
HIST = 50
EMBED_DIM = 64

NUM_CORES = 2
NUM_SUBCORES = 16
NUM_WORKERS = NUM_CORES * NUM_SUBCORES
ROWS_PER_WORKER = BATCH // NUM_WORKERS
IDX_PER_WORKER = ROWS_PER_WORKER * HIST


def _body(x_hbm, table_hbm, out_hbm, idx_v, sem):
    c = lax.axis_index("c")
    s = lax.axis_index("s")
    wid = s * NUM_CORES + c
    row0 = wid * ROWS_PER_WORKER

    pltpu.sync_copy(x_hbm.at[pl.ds(wid * IDX_PER_WORKER, IDX_PER_WORKER)], idx_v)

    def _row(i, carry):
        p0 = i * HIST
        for base, lo, hi in ((0, 0, 16), (16, 0, 16), (32, 0, 16), (34, 14, 16)):
            vec = idx_v[pl.ds(p0 + base, 16)]
            for k in range(lo, hi):
                r = vec[k] + 1
                pltpu.async_copy(
                    table_hbm.at[r], out_hbm.at[row0 + i, base + k], sem
                )
        return carry

    lax.fori_loop(0, ROWS_PER_WORKER, _row, 0)

    def _drain(i, carry):
        for _ in range(HIST):
            pltpu.make_async_copy(
                table_hbm.at[0], out_hbm.at[row0 + i, 0], sem
            ).wait()
        return carry

    lax.fori_loop(0, ROWS_PER_WORKER, _drain, 0)


def kernel(x, table):
    mesh = plsc.VectorSubcoreMesh(
        core_axis_name="c",
        subcore_axis_name="s",
        num_cores=NUM_CORES,
        num_subcores=NUM_SUBCORES,
    )
    return pl.kernel(
        _body,
        out_type=jax.ShapeDtypeStruct((BATCH, HIST, EMBED_DIM), jnp.float32),
        mesh=mesh,
        scratch_types=[
            pltpu.VMEM((IDX_PER_WORKER,), jnp.int32),
            pltpu.SemaphoreType.DMA,
        ],
    )(x.reshape(-1), table)


# speedup vs baseline: 4.2468x; 4.2468x over previous
import jax
import jax.numpy as jnp
from jax import lax
from jax.experimental import pallas as pl
from jax.experimental.pallas import tpu as pltpu
from jax.experimental.pallas import tpu_sc as plsc

BATCH = 4096
HIST = 50
EMBED_DIM = 64
VOCAB = 1000000

NUM_CORES = 2
NUM_SUBCORES = 16
NUM_WORKERS = NUM_CORES * NUM_SUBCORES
ROWS_PER_WORKER = BATCH // NUM_WORKERS


def _body(x_hbm, table_hbm, out_hbm, idx_v, rows0, rows1, gsem, osem):
    c = lax.axis_index("c")
    s = lax.axis_index("s")
    wid = s * NUM_CORES + c
    row0 = wid * ROWS_PER_WORKER

    shifted = table_hbm.at[pl.ds(1, VOCAB)]

    pltpu.sync_copy(x_hbm.at[pl.ds(row0, ROWS_PER_WORKER)], idx_v)

    pltpu.async_copy(shifted.at[idx_v.at[0]], rows0, gsem)

    def _pair(p, carry):
        i0 = 2 * p
        pltpu.make_async_copy(shifted.at[idx_v.at[i0]], rows0, gsem).wait()
        pltpu.async_copy(shifted.at[idx_v.at[i0 + 1]], rows1, gsem)
        pltpu.async_copy(rows0, out_hbm.at[row0 + i0], osem)
        pltpu.make_async_copy(shifted.at[idx_v.at[i0 + 1]], rows1, gsem).wait()
        pltpu.make_async_copy(rows0, out_hbm.at[row0 + i0], osem).wait()

        @pl.when(p + 1 < ROWS_PER_WORKER // 2)
        def _():
            pltpu.async_copy(shifted.at[idx_v.at[i0 + 2]], rows0, gsem)

        pltpu.async_copy(rows1, out_hbm.at[row0 + i0 + 1], osem)
        pltpu.make_async_copy(rows1, out_hbm.at[row0 + i0 + 1], osem).wait()
        return carry

    lax.fori_loop(0, ROWS_PER_WORKER // 2, _pair, 0)


def kernel(x, table):
    mesh = plsc.VectorSubcoreMesh(
        core_axis_name="c",
        subcore_axis_name="s",
        num_cores=NUM_CORES,
        num_subcores=NUM_SUBCORES,
    )
    return pl.kernel(
        _body,
        out_type=jax.ShapeDtypeStruct((BATCH, HIST, EMBED_DIM), jnp.float32),
        mesh=mesh,
        scratch_types=[
            pltpu.VMEM((ROWS_PER_WORKER, HIST), jnp.int32),
            pltpu.VMEM((HIST, EMBED_DIM), jnp.float32),
            pltpu.VMEM((HIST, EMBED_DIM), jnp.float32),
            pltpu.SemaphoreType.DMA,
            pltpu.SemaphoreType.DMA,
        ],
        compiler_params=pltpu.CompilerParams(use_tc_tiling_on_sc=False),
    )(x, table)


# speedup vs baseline: 4.4760x; 1.0540x over previous
import jax
import jax.numpy as jnp
from jax import lax
from jax.experimental import pallas as pl
from jax.experimental.pallas import tpu as pltpu
from jax.experimental.pallas import tpu_sc as plsc

BATCH = 4096
HIST = 50
EMBED_DIM = 64
VOCAB = 1000000

NUM_CORES = 2
NUM_SUBCORES = 16
NUM_WORKERS = NUM_CORES * NUM_SUBCORES
ROWS_PER_WORKER = BATCH // NUM_WORKERS
IDX_PER_WORKER = ROWS_PER_WORKER * HIST
CHUNK = 128
N_CHUNKS = IDX_PER_WORKER // CHUNK
LANE_PAD = 128


def _body(xp_hbm, table_hbm, out_hbm, xp_v, idx_c, rows0, rows1, gsem, osem):
    c = lax.axis_index("c")
    s = lax.axis_index("s")
    wid = s * NUM_CORES + c
    row0 = wid * ROWS_PER_WORKER
    out0 = wid * IDX_PER_WORKER

    shifted = table_hbm.at[pl.ds(1, VOCAB)]

    pltpu.sync_copy(xp_hbm.at[pl.ds(row0, ROWS_PER_WORKER)], xp_v)

    def _compact(i, carry):
        dst = i * HIST
        idx_c[pl.ds(dst, 16)] = xp_v[i, pl.ds(0, 16)]
        idx_c[pl.ds(dst + 16, 16)] = xp_v[i, pl.ds(16, 16)]
        idx_c[pl.ds(dst + 32, 16)] = xp_v[i, pl.ds(32, 16)]
        idx_c[pl.ds(dst + 34, 16)] = xp_v[i, pl.ds(34, 16)]
        return carry

    lax.fori_loop(0, ROWS_PER_WORKER, _compact, 0)

    pltpu.async_copy(shifted.at[idx_c.at[pl.ds(0, CHUNK)]], rows0, gsem)

    def _pair(p, carry):
        j0 = 2 * p * CHUNK
        pltpu.make_async_copy(
            shifted.at[idx_c.at[pl.ds(j0, CHUNK)]], rows0, gsem
        ).wait()
        pltpu.async_copy(
            shifted.at[idx_c.at[pl.ds(j0 + CHUNK, CHUNK)]], rows1, gsem
        )
        pltpu.async_copy(rows0, out_hbm.at[pl.ds(out0 + j0, CHUNK)], osem)
        pltpu.make_async_copy(
            shifted.at[idx_c.at[pl.ds(j0 + CHUNK, CHUNK)]], rows1, gsem
        ).wait()
        pltpu.make_async_copy(
            rows0, out_hbm.at[pl.ds(out0 + j0, CHUNK)], osem
        ).wait()

        @pl.when(p + 1 < N_CHUNKS // 2)
        def _():
            pltpu.async_copy(
                shifted.at[idx_c.at[pl.ds(j0 + 2 * CHUNK, CHUNK)]], rows0, gsem
            )

        pltpu.async_copy(
            rows1, out_hbm.at[pl.ds(out0 + j0 + CHUNK, CHUNK)], osem
        )
        pltpu.make_async_copy(
            rows1, out_hbm.at[pl.ds(out0 + j0 + CHUNK, CHUNK)], osem
        ).wait()
        return carry

    lax.fori_loop(0, N_CHUNKS // 2, _pair, 0)


def kernel(x, table):
    xp = lax.pad(x, jnp.int32(0), ((0, 0, 0), (0, LANE_PAD - HIST, 0)))
    mesh = plsc.VectorSubcoreMesh(
        core_axis_name="c",
        subcore_axis_name="s",
        num_cores=NUM_CORES,
        num_subcores=NUM_SUBCORES,
    )
    out = pl.kernel(
        _body,
        out_type=jax.ShapeDtypeStruct((BATCH * HIST, EMBED_DIM), jnp.float32),
        mesh=mesh,
        scratch_types=[
            pltpu.VMEM((ROWS_PER_WORKER, LANE_PAD), jnp.int32),
            pltpu.VMEM((IDX_PER_WORKER,), jnp.int32),
            pltpu.VMEM((CHUNK, EMBED_DIM), jnp.float32),
            pltpu.VMEM((CHUNK, EMBED_DIM), jnp.float32),
            pltpu.SemaphoreType.DMA,
            pltpu.SemaphoreType.DMA,
        ],
        compiler_params=pltpu.CompilerParams(use_tc_tiling_on_sc=False),
    )(xp, table)
    return out.reshape(BATCH, HIST, EMBED_DIM)
